# Initial kernel scaffold; baseline (speedup 1.0000x reference)
import jax
import jax.numpy as jnp
from jax.experimental import pallas as pl


def kernel(words, pos, W_words, W_pos):
    raise NotImplementedError("write your pallas kernel here")


# speedup vs baseline: 2.4555x; 2.4555x over previous
import functools

import jax
import jax.numpy as jnp
from jax import lax
from jax.experimental import pallas as pl
from jax.experimental.pallas import tpu as pltpu
from jax.experimental.pallas import tpu_sc as plsc

B, L = 4096, 200
DW, DP = 32, 32
DO = DW + DP
N = B * L
NC, NS = 2, 16
NW = NC * NS
PER_W = N // NW
G = 128
NG = PER_W // G


def _body(words_hbm, pos_hbm, ww_hbm, wp_hbm, out_hbm,
          idxw_v, idxp_v, rw_v, rp_v, semw, semp):
    wid = lax.axis_index("s") * NC + lax.axis_index("c")
    pltpu.sync_copy(words_hbm.at[wid], idxw_v)
    pltpu.sync_copy(pos_hbm.at[wid], idxp_v)
    base = wid * PER_W

    def step(j, carry):
        pltpu.async_copy(ww_hbm.at[idxw_v.at[j]], rw_v, semw).wait()
        pltpu.async_copy(wp_hbm.at[idxp_v.at[j]], rp_v, semp).wait()
        row0 = base + j * G
        pltpu.sync_copy(rw_v, out_hbm.at[pl.ds(row0, G), pl.ds(0, DW)])
        pltpu.sync_copy(rp_v, out_hbm.at[pl.ds(row0, G), pl.ds(DW, DP)])
        return carry

    lax.fori_loop(0, NG, step, 0)


@jax.jit
def _run(words_r, pos_r, W_words, W_pos):
    mesh = plsc.VectorSubcoreMesh(
        core_axis_name="c", subcore_axis_name="s",
        num_cores=NC, num_subcores=NS)
    f = pl.kernel(
        _body,
        out_type=jax.ShapeDtypeStruct((N, DO), jnp.float32),
        mesh=mesh,
        compiler_params=pltpu.CompilerParams(use_tc_tiling_on_sc=False),
        scratch_types=[
            pltpu.VMEM((NG, G), jnp.int32),
            pltpu.VMEM((NG, G), jnp.int32),
            pltpu.VMEM((G, DW), jnp.float32),
            pltpu.VMEM((G, DP), jnp.float32),
            pltpu.SemaphoreType.DMA,
            pltpu.SemaphoreType.DMA,
        ],
    )
    return f(words_r, pos_r, W_words, W_pos)


def kernel(words, pos, W_words, W_pos):
    words_r = words.astype(jnp.int32).reshape(NW, NG, G)
    pos_r = pos.astype(jnp.int32).reshape(NW, NG, G)
    out = _run(words_r, pos_r, W_words, W_pos)
    return out.reshape(B, L, DO)


# speedup vs baseline: 2.8296x; 1.1524x over previous
import jax
import jax.numpy as jnp
from jax import lax
from jax.experimental import pallas as pl
from jax.experimental.pallas import tpu as pltpu
from jax.experimental.pallas import tpu_sc as plsc

B, L = 4096, 200
DW, DP = 32, 32
DO = DW + DP
N = B * L
NC, NS = 2, 16
NW = NC * NS
PER_W = N // NW
G = 128
NG = PER_W // G
NBUF = 4


def _body(words_hbm, pos_hbm, ww_hbm, wp_hbm, out_hbm,
          idxw_v, idxp_v, rw_v, rp_v, semw, semp, semo):
    wid = lax.axis_index("s") * NC + lax.axis_index("c")
    pltpu.sync_copy(words_hbm.at[wid], idxw_v)
    pltpu.sync_copy(pos_hbm.at[wid], idxp_v)
    base = wid * PER_W

    def start_gather(b, j):
        pltpu.async_copy(ww_hbm.at[idxw_v.at[j]], rw_v.at[b], semw.at[b])
        pltpu.async_copy(wp_hbm.at[idxp_v.at[j]], rp_v.at[b], semp.at[b])

    def wait_gather(b, j):
        pltpu.make_async_copy(ww_hbm.at[idxw_v.at[j]], rw_v.at[b],
                              semw.at[b]).wait()
        pltpu.make_async_copy(wp_hbm.at[idxp_v.at[j]], rp_v.at[b],
                              semp.at[b]).wait()

    def start_write(b, j):
        row0 = base + j * G
        pltpu.async_copy(rw_v.at[b], out_hbm.at[pl.ds(row0, G), pl.ds(0, DW)],
                         semo.at[b])
        pltpu.async_copy(rp_v.at[b], out_hbm.at[pl.ds(row0, G), pl.ds(DW, DP)],
                         semo.at[b])

    def wait_write(b, j):
        row0 = base + j * G
        pltpu.make_async_copy(rw_v.at[b],
                              out_hbm.at[pl.ds(row0, G), pl.ds(0, DW)],
                              semo.at[b]).wait()
        pltpu.make_async_copy(rp_v.at[b],
                              out_hbm.at[pl.ds(row0, G), pl.ds(DW, DP)],
                              semo.at[b]).wait()

    for b in range(NBUF):
        start_gather(b, b)

    def step(it, carry):
        g = it * NBUF
        for b in range(NBUF):
            j = g + b
            wait_gather(b, j)
            start_write(b, j)
            wait_write(b, j)
            start_gather(b, j + NBUF)
        return carry

    lax.fori_loop(0, NG // NBUF - 1, step, 0)

    for b in range(NBUF):
        j = NG - NBUF + b
        wait_gather(b, j)
        start_write(b, j)
        wait_write(b, j)


@jax.jit
def _run(words_r, pos_r, W_words, W_pos):
    mesh = plsc.VectorSubcoreMesh(
        core_axis_name="c", subcore_axis_name="s",
        num_cores=NC, num_subcores=NS)
    f = pl.kernel(
        _body,
        out_type=jax.ShapeDtypeStruct((N, DO), jnp.float32),
        mesh=mesh,
        compiler_params=pltpu.CompilerParams(use_tc_tiling_on_sc=False),
        scratch_types=[
            pltpu.VMEM((NG, G), jnp.int32),
            pltpu.VMEM((NG, G), jnp.int32),
            pltpu.VMEM((NBUF, G, DW), jnp.float32),
            pltpu.VMEM((NBUF, G, DP), jnp.float32),
            pltpu.SemaphoreType.DMA((NBUF,)),
            pltpu.SemaphoreType.DMA((NBUF,)),
            pltpu.SemaphoreType.DMA((NBUF,)),
        ],
    )
    return f(words_r, pos_r, W_words, W_pos)


def kernel(words, pos, W_words, W_pos):
    words_r = words.astype(jnp.int32).reshape(NW, NG, G)
    pos_r = pos.astype(jnp.int32).reshape(NW, NG, G)
    out = _run(words_r, pos_r, W_words, W_pos)
    return out.reshape(B, L, DO)
